# baseline (speedup 1.0000x reference)
import functools

import jax
import jax.numpy as jnp
from jax import lax
from jax.experimental import pallas as pl
from jax.experimental.pallas import tpu as pltpu
from jax.experimental.pallas import tpu_sc as plsc

N_FIELDS = 26
VOCAB = 100000
DIM = 16
BATCH = 16384

NC = 2
NS = 16
NW = NC * NS
B_PER_W = BATCH // NW
CHUNK = 128
NCH = B_PER_W // CHUNK
FC = N_FIELDS * CHUNK
L = 16


@functools.partial(
    pl.kernel,
    mesh=plsc.VectorSubcoreMesh(core_axis_name="c", subcore_axis_name="s"),
    compiler_params=pltpu.CompilerParams(
        needs_layout_passes=False, use_tc_tiling_on_sc=False),
    out_type=jax.ShapeDtypeStruct((BATCH,), jnp.float32),
    scratch_types=[
        pltpu.VMEM((FC,), jnp.int32),
        pltpu.VMEM((FC, DIM), jnp.float32),
        pltpu.VMEM((FC,), jnp.float32),
        pltpu.VMEM((CHUNK,), jnp.float32),
        pltpu.SemaphoreType.DMA,
        pltpu.SemaphoreType.DMA,
    ],
)
def _fm_sc(idx_hbm, emb_hbm, lin_hbm, out_hbm,
           idx_v, emb_v, lin_v, outb_v, sem_e, sem_l):
    wid = lax.axis_index("s") * NC + lax.axis_index("c")
    lane = lax.iota(jnp.int32, L)

    for c in range(NCH):
        pltpu.sync_copy(idx_hbm.at[wid, c], idx_v)
        cp_e = pltpu.async_copy(emb_hbm.at[idx_v], emb_v, sem_e)
        cp_l = pltpu.async_copy(lin_hbm.at[idx_v], lin_v, sem_l)
        cp_e.wait()
        cp_l.wait()

        def group_body(g, _):
            s0 = g * L
            zero = jnp.zeros((L,), jnp.float32)

            def dim_body(d, carry):
                cross, ssq = carry
                dcol = jnp.full((L,), d, jnp.int32)
                sd = zero
                for f in range(N_FIELDS):
                    rows = f * CHUNK + s0 + lane
                    v = plsc.load_gather(emb_v, [rows, dcol])
                    sd = sd + v
                    ssq = ssq + v * v
                return cross + sd * sd, ssq

            cross, ssq = lax.fori_loop(0, DIM, dim_body, (zero, zero))
            res = (cross - ssq) * 0.5
            for f in range(N_FIELDS):
                res = res + lin_v[pl.ds(f * CHUNK + s0, L)]
            outb_v[pl.ds(s0, L)] = 1.0 / (1.0 + jnp.exp(-res))
            return 0

        lax.fori_loop(0, CHUNK // L, group_body, 0)
        pltpu.sync_copy(outb_v,
                        out_hbm.at[pl.ds(wid * B_PER_W + c * CHUNK, CHUNK)])


def kernel(x, emb_table, lin_table):
    offsets = (jnp.arange(N_FIELDS) * VOCAB).astype(x.dtype)
    flat = (x + offsets[None, :]).astype(jnp.int32)
    idx = flat.reshape(NW, NCH, CHUNK, N_FIELDS)
    idx = idx.transpose(0, 1, 3, 2).reshape(NW, NCH, FC)
    lin_flat = lin_table.reshape(-1)
    out = _fm_sc(idx, emb_table, lin_flat)
    return out.reshape(BATCH, 1)
